# baseline (speedup 1.0000x reference)
import functools

import jax
import jax.numpy as jnp
from jax import lax
from jax.experimental import pallas as pl
from jax.experimental.pallas import tpu as pltpu
from jax.experimental.pallas import tpu_sc as plsc

_K = 8
_E = 64
_NW = 32


def _logits_body(x_ref, w_ref, b_ref, out_ref):
    out_ref[...] = (
        jnp.dot(x_ref[...], w_ref[...], preferred_element_type=jnp.float32)
        + b_ref[...]
    )


def _tc_logits(x, W, b2, bm):
    m_total, hx = x.shape
    return pl.pallas_call(
        _logits_body,
        grid=(m_total // bm,),
        in_specs=[
            pl.BlockSpec((bm, hx), lambda i: (i, 0)),
            pl.BlockSpec((hx, _E), lambda i: (0, 0)),
            pl.BlockSpec((1, _E), lambda i: (0, 0)),
        ],
        out_specs=pl.BlockSpec((bm, _E), lambda i: (i, 0)),
        out_shape=jax.ShapeDtypeStruct((m_total, _E), jnp.float32),
        compiler_params=pltpu.CompilerParams(
            dimension_semantics=("arbitrary",),
        ),
    )(x, W, b2)


def _make_sc_router(m_total):
    rows_per_w = m_total // _NW
    groups = rows_per_w // 16
    mesh = plsc.VectorSubcoreMesh(core_axis_name="c", subcore_axis_name="s")

    @functools.partial(
        pl.kernel,
        mesh=mesh,
        compiler_params=pltpu.CompilerParams(needs_layout_passes=False),
        out_type=[
            jax.ShapeDtypeStruct((m_total * _E,), jnp.float32),
            jax.ShapeDtypeStruct((m_total * _K,), jnp.int32),
        ],
        scratch_types=[
            pltpu.VMEM((rows_per_w * _E,), jnp.float32),
            pltpu.VMEM((rows_per_w * _E,), jnp.float32),
            pltpu.VMEM((rows_per_w * _K,), jnp.int32),
        ],
    )
    def _route(logits_hbm, scores_hbm, idx_hbm, lbuf, sbuf, ibuf):
        wid = lax.axis_index("s") * 2 + lax.axis_index("c")
        base = wid * rows_per_w
        pltpu.sync_copy(logits_hbm.at[pl.ds(base * _E, rows_per_w * _E)], lbuf)

        lane64 = lax.iota(jnp.int32, 16) * _E
        lane8 = lax.iota(jnp.int32, 16) * _K
        neg_inf = jnp.float32(-jnp.inf)

        def group_body(g, carry):
            goff = g * (16 * _E) + lane64
            cols = [
                plsc.load_gather(lbuf, [goff + jnp.int32(e)])
                for e in range(_E)
            ]
            cur = list(cols)
            vals = []
            idxs = []
            for _ in range(_K):
                m = cur[0]
                for e in range(1, _E):
                    m = jnp.maximum(m, cur[e])
                idx = jnp.full((16,), _E, jnp.int32)
                for e in range(_E):
                    idx = jnp.minimum(
                        idx,
                        jnp.where(cur[e] == m, jnp.int32(e), jnp.int32(_E)),
                    )
                for e in range(_E):
                    cur[e] = jnp.where(idx == jnp.int32(e), neg_inf, cur[e])
                vals.append(m)
                idxs.append(idx)

            m0 = vals[0]
            denom = jnp.exp(vals[0] - m0)
            for v in vals[1:]:
                denom = denom + jnp.exp(v - m0)
            inv = 1.0 / denom
            for e in range(_E):
                score_e = jnp.where(
                    cur[e] == neg_inf,
                    jnp.exp(cols[e] - m0) * inv,
                    jnp.float32(0.0),
                )
                plsc.store_scatter(sbuf, [goff + jnp.int32(e)], score_e)
            ioff = g * (16 * _K) + lane8
            for k in range(_K):
                plsc.store_scatter(ibuf, [ioff + jnp.int32(k)], idxs[k])
            return carry

        lax.fori_loop(0, groups, group_body, 0)
        pltpu.sync_copy(sbuf, scores_hbm.at[pl.ds(base * _E, rows_per_w * _E)])
        pltpu.sync_copy(ibuf, idx_hbm.at[pl.ds(base * _K, rows_per_w * _K)])

    return _route


def kernel(hidden_states, W, b):
    Bx, Sx, Hx = hidden_states.shape
    m_total = Bx * Sx
    x = hidden_states.reshape(m_total, Hx)
    b2 = b.reshape(1, _E)

    bm = 2048 if m_total % 2048 == 0 else m_total
    logits = _tc_logits(x, W, b2, bm)
    scores_flat, idx_flat = _make_sc_router(m_total)(logits.reshape(-1))
    return scores_flat.reshape(m_total, _E), idx_flat.reshape(m_total, _K)


# speedup vs baseline: 2.4174x; 2.4174x over previous
import jax
import jax.numpy as jnp
from jax import lax
from jax.experimental import pallas as pl
from jax.experimental.pallas import tpu as pltpu

_K = 8


def _router_body(x_ref, w_ref, b_ref, scores_ref, idx_ref):
    w = w_ref[...]
    half = x_ref.shape[0] // 2
    for h in range(2):
        x = x_ref[h * half:(h + 1) * half, :]
        _route_half(
            x, w, b_ref,
            scores_ref.at[h * half:(h + 1) * half, :],
            idx_ref.at[h * half:(h + 1) * half, :],
        )


def _route_half(x, w, b_ref, scores_ref, idx_ref):
    logits = jnp.dot(x, w, preferred_element_type=jnp.float32) + b_ref[...]
    lt = logits.T

    n_exp, bm = lt.shape
    iota_e = lax.broadcasted_iota(jnp.int32, (n_exp, bm), 0).astype(jnp.float32)
    neg_inf = jnp.float32(-jnp.inf)

    cur = lt
    vals = []
    idxs = []
    for _ in range(_K):
        m = jnp.max(cur, axis=0, keepdims=True)
        at_max = cur == m
        idx = jnp.min(
            jnp.where(at_max, iota_e, jnp.float32(n_exp)), axis=0, keepdims=True
        )
        cur = jnp.where(iota_e == idx, neg_inf, cur)
        vals.append(m)
        idxs.append(idx)

    chosen = cur == neg_inf
    m0 = vals[0]
    denom = jnp.exp(vals[0] - m0)
    for v in vals[1:]:
        denom = denom + jnp.exp(v - m0)
    inv = 1.0 / denom
    scores_t = jnp.where(chosen, jnp.exp(lt - m0) * inv, jnp.float32(0.0))
    scores_ref[...] = scores_t.T

    idx_t = jnp.concatenate(idxs, axis=0)
    idx_ref[...] = idx_t.T.astype(jnp.int32)


def kernel(hidden_states, W, b):
    Bx, Sx, Hx = hidden_states.shape
    n_exp = W.shape[1]
    m_total = Bx * Sx
    x = hidden_states.reshape(m_total, Hx)
    b2 = b.reshape(1, n_exp)

    bm = 2048 if m_total % 2048 == 0 else m_total
    grid = (m_total // bm,)

    scores, indices = pl.pallas_call(
        _router_body,
        grid=grid,
        in_specs=[
            pl.BlockSpec((bm, Hx), lambda i: (i, 0)),
            pl.BlockSpec((Hx, n_exp), lambda i: (0, 0)),
            pl.BlockSpec((1, n_exp), lambda i: (0, 0)),
        ],
        out_specs=[
            pl.BlockSpec((bm, n_exp), lambda i: (i, 0)),
            pl.BlockSpec((bm, _K), lambda i: (i, 0)),
        ],
        out_shape=[
            jax.ShapeDtypeStruct((m_total, n_exp), jnp.float32),
            jax.ShapeDtypeStruct((m_total, _K), jnp.int32),
        ],
        compiler_params=pltpu.CompilerParams(
            dimension_semantics=("arbitrary",),
        ),
    )(x, W, b2)
    return scores, indices
